# baseline (speedup 1.0000x reference)
import functools

import jax
import jax.numpy as jnp
from jax import lax
from jax.experimental import pallas as pl
from jax.experimental.pallas import tpu as pltpu
from jax.experimental.pallas import tpu_sc as plsc

BATCH = 16384
EMB_DIM = 32
LANES = 16
NUM_WORKERS = 32
B_PER_W = BATCH // NUM_WORKERS
GROUPS = B_PER_W // LANES


def _dot_sigmoid_kernel(user_ids_hbm, item_ids_hbm, user_emb_hbm,
                        item_emb_hbm, out_hbm,
                        uidx_v, iidx_v, urows_v, irows_v, out_v, sem):
    nc = 2
    wid = lax.axis_index("s") * nc + lax.axis_index("c")
    base = wid * B_PER_W

    pltpu.sync_copy(user_ids_hbm.at[pl.ds(base, B_PER_W)], uidx_v)
    pltpu.sync_copy(item_ids_hbm.at[pl.ds(base, B_PER_W)], iidx_v)

    cp_u = pltpu.make_async_copy(user_emb_hbm.at[uidx_v], urows_v, sem)
    cp_i = pltpu.make_async_copy(item_emb_hbm.at[iidx_v], irows_v, sem)
    cp_u.start()
    cp_i.start()
    cp_u.wait()
    cp_i.wait()

    lane_iota = lax.iota(jnp.int32, LANES)

    def group_body(g, carry):
        row_idx = g * LANES + lane_iota
        acc = jnp.zeros((LANES,), jnp.float32)
        for k in range(EMB_DIM):
            col_idx = jnp.full((LANES,), k, jnp.int32)
            uv = plsc.load_gather(urows_v, [row_idx, col_idx])
            iv = plsc.load_gather(irows_v, [row_idx, col_idx])
            acc = acc + uv * iv
        sig = 1.0 / (1.0 + jnp.exp(-acc))
        out_v[pl.ds(g * LANES, LANES)] = sig
        return carry

    lax.fori_loop(0, GROUPS, group_body, 0)

    pltpu.sync_copy(out_v, out_hbm.at[pl.ds(base, B_PER_W)])


@jax.jit
def _run(user_ids, item_ids, user_emb, item_emb):
    mesh = plsc.VectorSubcoreMesh(core_axis_name="c", subcore_axis_name="s")
    kfn = functools.partial(
        pl.kernel,
        mesh=mesh,
        out_type=jax.ShapeDtypeStruct((BATCH,), jnp.float32),
        scratch_types=[
            pltpu.VMEM((B_PER_W,), jnp.int32),
            pltpu.VMEM((B_PER_W,), jnp.int32),
            pltpu.VMEM((B_PER_W, EMB_DIM), jnp.float32),
            pltpu.VMEM((B_PER_W, EMB_DIM), jnp.float32),
            pltpu.VMEM((B_PER_W,), jnp.float32),
            pltpu.SemaphoreType.DMA,
        ],
        compiler_params=pltpu.CompilerParams(
            needs_layout_passes=False, use_tc_tiling_on_sc=False),
    )(_dot_sigmoid_kernel)
    return kfn(user_ids, item_ids, user_emb, item_emb)


def kernel(user_ids, item_ids, user_emb, item_emb):
    return _run(user_ids.astype(jnp.int32), item_ids.astype(jnp.int32),
                user_emb, item_emb)
